# baseline (speedup 1.0000x reference)
import functools

import jax
import jax.numpy as jnp
import numpy as np
from jax.experimental import pallas as pl
from jax.experimental.pallas import tpu as pltpu

IN_F = 2048
OUT_F = 2048
E = 8
TOP_K = 2
RANK = 16
ALPHA = 1.0 / RANK
CAP_FACTOR = 1.5

BG = 1024
BT = 512


def _gate_kernel(x_ref, gw1_ref, la_ref, gb1_ref, lns_ref, lnb_ref, gw2_ref,
                 gb2_ref, w0_ref, w1_ref, cnt_ref, t_ref):
    x = x_ref[...]
    h = jax.lax.dot_general(gw1_ref[...], x, (((1,), (1,)), ((), ())),
                            preferred_element_type=jnp.float32) + gb1_ref[...]
    for e in range(E):
        te = jax.lax.dot_general(la_ref[e], x, (((0,), (1,)), ((), ())),
                                 preferred_element_type=jnp.float32)
        t_ref[e * RANK:(e + 1) * RANK, :] = te.astype(jnp.bfloat16)
    mu = jnp.mean(h, axis=0, keepdims=True)
    var = jnp.mean((h - mu) ** 2, axis=0, keepdims=True)
    h = (h - mu) / jnp.sqrt(var + 1e-5) * lns_ref[...] + lnb_ref[...]
    h = jnp.maximum(h, 0.0)
    g = jax.lax.dot_general(gw2_ref[...], h, (((1,), (0,)), ((), ())),
                            preferred_element_type=jnp.float32) + gb2_ref[...]

    iota = jax.lax.broadcasted_iota(jnp.int32, g.shape, 0)
    m1 = jnp.max(g, axis=0, keepdims=True)
    i1 = jnp.min(jnp.where(g == m1, iota, E), axis=0, keepdims=True)
    gm = jnp.where(iota == i1, -jnp.inf, g)
    m2 = jnp.max(gm, axis=0, keepdims=True)
    i2 = jnp.min(jnp.where(gm == m2, iota, E), axis=0, keepdims=True)

    e2 = jnp.exp(m2 - m1)
    p1 = 1.0 / (1.0 + e2)
    p2 = e2 / (1.0 + e2)

    oh1 = (iota == i1).astype(jnp.float32)
    oh2 = (iota == i2).astype(jnp.float32)
    w0_ref[...] = oh1 * p1
    w1_ref[...] = oh2 * p2
    c0 = jnp.sum(oh1, axis=1, keepdims=True)
    c1 = jnp.sum(oh2, axis=1, keepdims=True)
    cnt_ref[...] = jnp.concatenate([c0, c1], axis=1)[None]


def _main_kernel(x_ref, wt_ref, lb_ref, r_ref, t_ref,
                 w0_ref, w1_ref, cnt_ref, out_ref, *, capacity):
    x = x_ref[...].astype(jnp.bfloat16)
    acc = jax.lax.dot_general(x, wt_ref[...], (((1,), (1,)), ((), ())),
                              preferred_element_type=jnp.float32)
    counts = jnp.sum(cnt_ref[...], axis=0)
    allowed = (counts <= capacity).astype(jnp.float32)
    w = (w0_ref[...] * allowed[:, 0:1]
         + w1_ref[...] * allowed[:, 1:2])
    wexp = jax.lax.dot_general(r_ref[...], w, (((0,), (0,)), ((), ())),
                               preferred_element_type=jnp.float32,
                               precision=jax.lax.Precision.HIGHEST)
    u = (t_ref[...].astype(jnp.float32) * wexp * ALPHA).astype(jnp.bfloat16)
    bcat = lb_ref[...].reshape(E * RANK, OUT_F).astype(jnp.bfloat16)
    acc = acc + jax.lax.dot_general(u, bcat, (((0,), (0,)), ((), ())),
                                    preferred_element_type=jnp.float32)
    out_ref[...] = acc


def kernel(x, weight, lora_A, lora_B, gw1, gb1, ln_s, ln_b, gw2, gb2):
    tokens = x.shape[0]
    capacity = float(int(CAP_FACTOR * tokens / E))
    nbg = tokens // BG
    nbt = tokens // BT

    rexp = jnp.repeat(jnp.eye(E, dtype=jnp.float32), RANK, axis=1)

    w0, w1, cnt, t = pl.pallas_call(
        _gate_kernel,
        grid=(nbg,),
        in_specs=[
            pl.BlockSpec((BG, IN_F), lambda i: (i, 0)),
            pl.BlockSpec((4 * E, IN_F), lambda i: (0, 0)),
            pl.BlockSpec((E, IN_F, RANK), lambda i: (0, 0, 0)),
            pl.BlockSpec((4 * E, 1), lambda i: (0, 0)),
            pl.BlockSpec((4 * E, 1), lambda i: (0, 0)),
            pl.BlockSpec((4 * E, 1), lambda i: (0, 0)),
            pl.BlockSpec((E, 4 * E), lambda i: (0, 0)),
            pl.BlockSpec((E, 1), lambda i: (0, 0)),
        ],
        out_specs=[
            pl.BlockSpec((E, BG), lambda i: (0, i)),
            pl.BlockSpec((E, BG), lambda i: (0, i)),
            pl.BlockSpec((1, E, TOP_K), lambda i: (i, 0, 0)),
            pl.BlockSpec((E * RANK, BG), lambda i: (0, i)),
        ],
        out_shape=[
            jax.ShapeDtypeStruct((E, tokens), jnp.float32),
            jax.ShapeDtypeStruct((E, tokens), jnp.float32),
            jax.ShapeDtypeStruct((nbg, E, TOP_K), jnp.float32),
            jax.ShapeDtypeStruct((E * RANK, tokens), jnp.bfloat16),
        ],
        compiler_params=pltpu.CompilerParams(
            dimension_semantics=("parallel",)),
    )(x, gw1, lora_A, gb1[:, None], ln_s[:, None], ln_b[:, None], gw2,
      gb2[:, None])

    out = pl.pallas_call(
        functools.partial(_main_kernel, capacity=capacity),
        grid=(nbt,),
        in_specs=[
            pl.BlockSpec((BT, IN_F), lambda i: (i, 0)),
            pl.BlockSpec((OUT_F, IN_F), lambda i: (0, 0)),
            pl.BlockSpec((E, RANK, OUT_F), lambda i: (0, 0, 0)),
            pl.BlockSpec((E, E * RANK), lambda i: (0, 0)),
            pl.BlockSpec((E * RANK, BT), lambda i: (0, i)),
            pl.BlockSpec((E, BT), lambda i: (0, i)),
            pl.BlockSpec((E, BT), lambda i: (0, i)),
            pl.BlockSpec((nbg, E, TOP_K), lambda i: (0, 0, 0)),
        ],
        out_specs=pl.BlockSpec((BT, OUT_F), lambda i: (i, 0)),
        out_shape=jax.ShapeDtypeStruct((tokens, OUT_F), jnp.float32),
        compiler_params=pltpu.CompilerParams(
            dimension_semantics=("parallel",)),
    )(x, weight.astype(jnp.bfloat16), lora_B, rexp, t, w0, w1, cnt)
    return out


# speedup vs baseline: 1.4085x; 1.4085x over previous
import jax
import jax.numpy as jnp
import numpy as np
from jax.experimental import pallas as pl
from jax.experimental.pallas import tpu as pltpu

IN_F = 2048
OUT_F = 2048
E = 8
TOP_K = 2
RANK = 16
ALPHA = 1.0 / RANK
CAP_FACTOR = 1.5

BG = 1024
BT = 512


def _gate_kernel(x_ref, wg_ref, gb1_ref, lns_ref, lnb_ref, gw2t_ref, gb2_ref,
                 w0_ref, w1_ref, cnt_ref, t_ref):
    ht = jax.lax.dot_general(wg_ref[...], x_ref[...], (((0,), (1,)), ((), ())),
                             preferred_element_type=jnp.float32)
    h = ht[:4 * E, :] + gb1_ref[...]
    t_ref[...] = ht[4 * E:, :].astype(jnp.bfloat16)
    mu = jnp.mean(h, axis=0, keepdims=True)
    var = jnp.mean((h - mu) ** 2, axis=0, keepdims=True)
    h = (h - mu) / jnp.sqrt(var + 1e-5) * lns_ref[...] + lnb_ref[...]
    h = jnp.maximum(h, 0.0)
    g = jax.lax.dot_general(gw2t_ref[...], h, (((0,), (0,)), ((), ())),
                            preferred_element_type=jnp.float32) + gb2_ref[...]

    iota = jax.lax.broadcasted_iota(jnp.int32, g.shape, 0)
    m1 = jnp.max(g, axis=0, keepdims=True)
    i1 = jnp.min(jnp.where(g == m1, iota, E), axis=0, keepdims=True)
    gm = jnp.where(iota == i1, -jnp.inf, g)
    m2 = jnp.max(gm, axis=0, keepdims=True)
    i2 = jnp.min(jnp.where(gm == m2, iota, E), axis=0, keepdims=True)

    e2 = jnp.exp(m2 - m1)
    p1 = 1.0 / (1.0 + e2)
    p2 = e2 / (1.0 + e2)

    oh1 = (iota == i1).astype(jnp.float32)
    oh2 = (iota == i2).astype(jnp.float32)
    w0_ref[...] = oh1 * p1
    w1_ref[...] = oh2 * p2
    c0 = jnp.sum(oh1, axis=1, keepdims=True)
    c1 = jnp.sum(oh2, axis=1, keepdims=True)
    cnt_ref[...] = jnp.concatenate([c0, c1], axis=1)[None]


def _main_kernel(x_ref, wt_ref, bcat_ref, r_ref, t_ref,
                 w0_ref, w1_ref, cnt_ref, cap_ref, out_ref):
    x = x_ref[...].astype(jnp.bfloat16)
    acc = jax.lax.dot_general(x, wt_ref[...], (((1,), (1,)), ((), ())),
                              preferred_element_type=jnp.float32)
    counts = jnp.sum(cnt_ref[...], axis=0)
    allowed = (counts <= cap_ref[...]).astype(jnp.float32)
    w = (w0_ref[...] * allowed[:, 0:1]
         + w1_ref[...] * allowed[:, 1:2])
    wexp = jax.lax.dot_general(r_ref[...], w, (((0,), (0,)), ((), ())),
                               preferred_element_type=jnp.float32,
                               precision=jax.lax.Precision.HIGHEST)
    u = (t_ref[...].astype(jnp.float32) * wexp * ALPHA).astype(jnp.bfloat16)
    acc = acc + jax.lax.dot_general(u, bcat_ref[...], (((0,), (0,)), ((), ())),
                                    preferred_element_type=jnp.float32)
    out_ref[...] = acc


def kernel(x, weight, lora_A, lora_B, gw1, gb1, ln_s, ln_b, gw2, gb2):
    tokens = x.shape[0]
    capacity = float(int(CAP_FACTOR * tokens / E))
    nbg = tokens // BG
    nbt = tokens // BT

    acat = lora_A.transpose(1, 0, 2).reshape(IN_F, E * RANK)
    wg = jnp.concatenate([gw1.T, acat], axis=1)
    bcat = lora_B.reshape(E * RANK, OUT_F).astype(jnp.bfloat16)
    rexp = jnp.repeat(jnp.eye(E, dtype=jnp.float32), RANK, axis=1)
    cap = jnp.full((E, TOP_K), capacity, dtype=jnp.float32)

    w0, w1, cnt, t = pl.pallas_call(
        _gate_kernel,
        grid=(nbg,),
        in_specs=[
            pl.BlockSpec((BG, IN_F), lambda i: (i, 0)),
            pl.BlockSpec((IN_F, 4 * E + E * RANK), lambda i: (0, 0)),
            pl.BlockSpec((4 * E, 1), lambda i: (0, 0)),
            pl.BlockSpec((4 * E, 1), lambda i: (0, 0)),
            pl.BlockSpec((4 * E, 1), lambda i: (0, 0)),
            pl.BlockSpec((4 * E, E), lambda i: (0, 0)),
            pl.BlockSpec((E, 1), lambda i: (0, 0)),
        ],
        out_specs=[
            pl.BlockSpec((E, BG), lambda i: (0, i)),
            pl.BlockSpec((E, BG), lambda i: (0, i)),
            pl.BlockSpec((1, E, TOP_K), lambda i: (i, 0, 0)),
            pl.BlockSpec((E * RANK, BG), lambda i: (0, i)),
        ],
        out_shape=[
            jax.ShapeDtypeStruct((E, tokens), jnp.float32),
            jax.ShapeDtypeStruct((E, tokens), jnp.float32),
            jax.ShapeDtypeStruct((nbg, E, TOP_K), jnp.float32),
            jax.ShapeDtypeStruct((E * RANK, tokens), jnp.bfloat16),
        ],
        compiler_params=pltpu.CompilerParams(
            dimension_semantics=("parallel",)),
    )(x, wg, gb1[:, None], ln_s[:, None], ln_b[:, None], gw2.T, gb2[:, None])

    out = pl.pallas_call(
        _main_kernel,
        grid=(nbt,),
        in_specs=[
            pl.BlockSpec((BT, IN_F), lambda i: (i, 0)),
            pl.BlockSpec((OUT_F, IN_F), lambda i: (0, 0)),
            pl.BlockSpec((E * RANK, OUT_F), lambda i: (0, 0)),
            pl.BlockSpec((E, E * RANK), lambda i: (0, 0)),
            pl.BlockSpec((E * RANK, BT), lambda i: (0, i)),
            pl.BlockSpec((E, BT), lambda i: (0, i)),
            pl.BlockSpec((E, BT), lambda i: (0, i)),
            pl.BlockSpec((nbg, E, TOP_K), lambda i: (0, 0, 0)),
            pl.BlockSpec((E, TOP_K), lambda i: (0, 0)),
        ],
        out_specs=pl.BlockSpec((BT, OUT_F), lambda i: (i, 0)),
        out_shape=jax.ShapeDtypeStruct((tokens, OUT_F), jnp.float32),
        compiler_params=pltpu.CompilerParams(
            dimension_semantics=("parallel",)),
    )(x, weight.astype(jnp.bfloat16), bcat, rexp, t, w0, w1, cnt, cap)
    return out
